# baseline (speedup 1.0000x reference)
import functools

import jax
import jax.numpy as jnp
from jax import lax
from jax.experimental import pallas as pl
from jax.experimental.pallas import tpu as pltpu
from jax.experimental.pallas import tpu_sc as plsc

ACT_DIM = 18
D_EMBED = 64
NPAIR = ACT_DIM * ACT_DIM
DP = 2 * D_EMBED

NUM_CORES = 2
NUM_SUBCORES = 16
NW = NUM_CORES * NUM_SUBCORES

CH = 512
PPG = 128


def _tanh16(v):
  return 1.0 - 2.0 / (jnp.exp(2.0 * v) + 1.0)


def _mesh():
  return plsc.VectorSubcoreMesh(
      core_axis_name="c", subcore_axis_name="s",
      num_cores=NUM_CORES, num_subcores=NUM_SUBCORES)


@jax.jit
def _sc_pair_table(table_flat):
  @functools.partial(
      pl.kernel,
      out_type=jax.ShapeDtypeStruct((NPAIR * DP,), jnp.float32),
      mesh=_mesh(),
      compiler_params=pltpu.CompilerParams(needs_layout_passes=False),
      scratch_types=[
          pltpu.VMEM((ACT_DIM * D_EMBED,), jnp.float32),
          pltpu.VMEM((NPAIR * DP,), jnp.float32),
      ],
  )
  def k(table_hbm, out_hbm, tblv, pairv):
    cid = lax.axis_index("c")
    sid = lax.axis_index("s")
    wid = sid * NUM_CORES + cid

    @pl.when(wid == 0)
    def _():
      pltpu.sync_copy(table_hbm, tblv)
      for i in range(ACT_DIM * D_EMBED // 16):
        sl = pl.ds(i * 16, 16)
        tblv[sl] = _tanh16(tblv[sl])

      def pair_body(a, carry):
        def inner(b, carry2):
          p = a * ACT_DIM + b
          for t in range(D_EMBED // 16):
            pairv[pl.ds(p * DP + t * 16, 16)] = (
                tblv[pl.ds(a * D_EMBED + t * 16, 16)])
            pairv[pl.ds(p * DP + D_EMBED + t * 16, 16)] = (
                tblv[pl.ds(b * D_EMBED + t * 16, 16)])
          return carry2
        return lax.fori_loop(0, ACT_DIM, inner, carry)

      lax.fori_loop(0, ACT_DIM, pair_body, 0)
      pltpu.sync_copy(pairv, out_hbm)

  return k(table_flat)


@functools.partial(jax.jit, static_argnames=("n_chunks",))
def _sc_embed(acts_flat, pair_table, n_chunks):
  n = acts_flat.shape[0]

  @functools.partial(
      pl.kernel,
      out_type=jax.ShapeDtypeStruct((n // 2, DP), jnp.float32),
      mesh=_mesh(),
      compiler_params=pltpu.CompilerParams(needs_layout_passes=False),
      scratch_types=[
          pltpu.VMEM((CH,), jnp.int32),
          pltpu.VMEM((CH // 2,), jnp.int32),
          pltpu.VMEM((CH // 2, DP), jnp.float32),
          pltpu.SemaphoreType.DMA,
      ],
  )
  def k(pair_hbm, acts_hbm, out_hbm, idxv, pidxv, rowsv, sem):
    cid = lax.axis_index("c")
    sid = lax.axis_index("s")
    wid = sid * NUM_CORES + cid
    base = wid * (n_chunks * CH)
    lane = lax.iota(jnp.int32, 16)

    def chunk_body(g, carry):
      off = base + g * CH
      pltpu.sync_copy(acts_hbm.at[pl.ds(off, CH)], idxv)
      for grp in range(CH // 32):
        ev = plsc.load_gather(idxv, [grp * 32 + 2 * lane])
        od = plsc.load_gather(idxv, [grp * 32 + 2 * lane + 1])
        pidxv[pl.ds(grp * 16, 16)] = ev * ACT_DIM + od
      copies = []
      for j in range(CH // 2 // PPG):
        sl = pl.ds(j * PPG, PPG)
        copies.append(
            pltpu.async_copy(pair_hbm.at[pidxv.at[sl]], rowsv.at[sl], sem))
      for c in copies:
        c.wait()
      off2 = pl.multiple_of(off // 2, 8)
      pltpu.sync_copy(rowsv, out_hbm.at[pl.ds(off2, CH // 2)])
      return carry

    lax.fori_loop(0, n_chunks, chunk_body, 0)

  return k(pair_table, acts_flat)


def kernel(acts, table):
  b, h = acts.shape
  n = b * h
  assert n % (NW * CH) == 0
  n_chunks = n // (NW * CH)
  acts_flat = acts.reshape(n).astype(jnp.int32)
  pair_tbl = _sc_pair_table(table.reshape(-1)).reshape(NPAIR, DP)
  out = _sc_embed(acts_flat, pair_tbl, n_chunks)
  return out.reshape(b, h, D_EMBED)


# speedup vs baseline: 1.3130x; 1.3130x over previous
import functools

import jax
import jax.numpy as jnp
from jax import lax
from jax.experimental import pallas as pl
from jax.experimental.pallas import tpu as pltpu
from jax.experimental.pallas import tpu_sc as plsc

ACT_DIM = 18
D_EMBED = 64
NPAIR = ACT_DIM * ACT_DIM
DP = 2 * D_EMBED

NUM_CORES = 2
NUM_SUBCORES = 16
NW = NUM_CORES * NUM_SUBCORES

CH = 1024
PPG = 128


def _tanh16(v):
  return 1.0 - 2.0 / (jnp.exp(2.0 * v) + 1.0)


def _mesh():
  return plsc.VectorSubcoreMesh(
      core_axis_name="c", subcore_axis_name="s",
      num_cores=NUM_CORES, num_subcores=NUM_SUBCORES)


@jax.jit
def _sc_pair_table(table_flat):
  @functools.partial(
      pl.kernel,
      out_type=jax.ShapeDtypeStruct((NPAIR * DP,), jnp.float32),
      mesh=_mesh(),
      compiler_params=pltpu.CompilerParams(needs_layout_passes=False),
      scratch_types=[
          pltpu.VMEM((ACT_DIM * D_EMBED,), jnp.float32),
          pltpu.VMEM((NPAIR * DP,), jnp.float32),
      ],
  )
  def k(table_hbm, out_hbm, tblv, pairv):
    cid = lax.axis_index("c")
    sid = lax.axis_index("s")
    wid = sid * NUM_CORES + cid

    @pl.when(wid == 0)
    def _():
      pltpu.sync_copy(table_hbm, tblv)
      for i in range(ACT_DIM * D_EMBED // 16):
        sl = pl.ds(i * 16, 16)
        tblv[sl] = _tanh16(tblv[sl])

      def pair_body(a, carry):
        def inner(b, carry2):
          p = a * ACT_DIM + b
          for t in range(D_EMBED // 16):
            pairv[pl.ds(p * DP + t * 16, 16)] = (
                tblv[pl.ds(a * D_EMBED + t * 16, 16)])
            pairv[pl.ds(p * DP + D_EMBED + t * 16, 16)] = (
                tblv[pl.ds(b * D_EMBED + t * 16, 16)])
          return carry2
        return lax.fori_loop(0, ACT_DIM, inner, carry)

      lax.fori_loop(0, ACT_DIM, pair_body, 0)
      pltpu.sync_copy(pairv, out_hbm)

  return k(table_flat)


@functools.partial(jax.jit, static_argnames=("n_chunks",))
def _sc_embed(acts_flat, pair_table, n_chunks):
  n = acts_flat.shape[0]

  @functools.partial(
      pl.kernel,
      out_type=jax.ShapeDtypeStruct((n // 2, DP), jnp.float32),
      mesh=_mesh(),
      compiler_params=pltpu.CompilerParams(needs_layout_passes=False),
      scratch_types=[
          pltpu.VMEM((CH,), jnp.int32),
          pltpu.VMEM((CH // 2,), jnp.int32),
          pltpu.VMEM((CH // 2, DP), jnp.float32),
          pltpu.VMEM_SHARED((NPAIR, DP), jnp.float32),
          pltpu.SemaphoreType.DMA,
      ],
  )
  def k(pair_hbm, acts_hbm, out_hbm, idxv, pidxv, rowsv, pair_sp, sem):
    cid = lax.axis_index("c")
    sid = lax.axis_index("s")
    wid = sid * NUM_CORES + cid
    base = wid * (n_chunks * CH)
    lane = lax.iota(jnp.int32, 16)

    @pl.when(sid == 0)
    def _():
      pltpu.sync_copy(pair_hbm, pair_sp)
    plsc.subcore_barrier()

    def chunk_body(g, carry):
      off = base + g * CH
      pltpu.sync_copy(acts_hbm.at[pl.ds(off, CH)], idxv)
      for grp in range(CH // 32):
        ev = plsc.load_gather(idxv, [grp * 32 + 2 * lane])
        od = plsc.load_gather(idxv, [grp * 32 + 2 * lane + 1])
        pidxv[pl.ds(grp * 16, 16)] = ev * ACT_DIM + od
      copies = []
      for j in range(CH // 2 // PPG):
        sl = pl.ds(j * PPG, PPG)
        copies.append(
            pltpu.async_copy(pair_sp.at[pidxv.at[sl]], rowsv.at[sl], sem))
      for c in copies:
        c.wait()
      off2 = pl.multiple_of(off // 2, 8)
      pltpu.sync_copy(rowsv, out_hbm.at[pl.ds(off2, CH // 2)])
      return carry

    lax.fori_loop(0, n_chunks, chunk_body, 0)

  return k(pair_table, acts_flat)


def kernel(acts, table):
  b, h = acts.shape
  n = b * h
  assert n % (NW * CH) == 0
  n_chunks = n // (NW * CH)
  acts_flat = acts.reshape(n).astype(jnp.int32)
  pair_tbl = _sc_pair_table(table.reshape(-1)).reshape(NPAIR, DP)
  out = _sc_embed(acts_flat, pair_tbl, n_chunks)
  return out.reshape(b, h, D_EMBED)
